# Initial kernel scaffold; baseline (speedup 1.0000x reference)
import jax
import jax.numpy as jnp
from jax.experimental import pallas as pl


def kernel(hidden_states, Wk, Wv, Wq, Wsf, gate_w, up_w, down_w):
    raise NotImplementedError("write your pallas kernel here")


# speedup vs baseline: 1.2782x; 1.2782x over previous
import jax
import jax.numpy as jnp
from jax.experimental import pallas as pl
from jax.experimental.pallas import tpu as pltpu


def _proj_kernel(x_ref, wk_ref, wv_ref, wsf_ref, wq_ref, k_ref, v_ref, q_ref):
    x = x_ref[...]
    k_ref[...] = jnp.dot(x, wk_ref[...], preferred_element_type=jnp.float32)
    v_ref[...] = jnp.dot(x, wv_ref[...], preferred_element_type=jnp.float32)
    sf = jnp.dot(x, wsf_ref[...], preferred_element_type=jnp.float32)
    q_ref[...] = jnp.dot(sf, wq_ref[...], preferred_element_type=jnp.float32)


def _moe_kernel(x_ref, gw_ref, uw_ref, dw_ref, scale_ref, out_ref, x_save_ref):
    e = pl.program_id(1)
    f = pl.program_id(2)

    @pl.when((e == 0) & (f == 0))
    def _():
        out_ref[...] = x_ref[...]

    @pl.when(f == 0)
    def _():
        x_save_ref[...] = out_ref[...]

    xb = x_save_ref[...].astype(jnp.bfloat16)
    g = jnp.dot(xb, gw_ref[0], preferred_element_type=jnp.float32)
    u = jnp.dot(xb, uw_ref[0], preferred_element_type=jnp.float32)
    h = (jax.nn.silu(g) * u).astype(jnp.bfloat16)
    o = jnp.dot(h, dw_ref[0], preferred_element_type=jnp.float32)
    out_ref[...] = out_ref[...] + o * scale_ref[0]


def kernel(hidden_states, Wk, Wv, Wq, Wsf, gate_w, up_w, down_w):
    b, s, d = hidden_states.shape
    n = b * s
    e_num, _, ff = gate_w.shape
    ea = Wk.shape[1]
    a_sz = ea // e_num
    hs = hidden_states.reshape(n, d)

    tp = min(512, n)
    keys, values, q = pl.pallas_call(
        _proj_kernel,
        grid=(n // tp,),
        in_specs=[
            pl.BlockSpec((tp, d), lambda t: (t, 0)),
            pl.BlockSpec((d, ea), lambda t: (0, 0)),
            pl.BlockSpec((d, ea), lambda t: (0, 0)),
            pl.BlockSpec((d, ea), lambda t: (0, 0)),
            pl.BlockSpec((ea, ea), lambda t: (0, 0)),
        ],
        out_specs=[pl.BlockSpec((tp, ea), lambda t: (t, 0))] * 3,
        out_shape=[jax.ShapeDtypeStruct((n, ea), jnp.float32)] * 3,
    )(hs, Wk, Wv, Wsf, Wq)

    qr = q.reshape(n, e_num, a_sz)
    k = keys.reshape(n, a_sz, e_num)
    qk = jnp.einsum('nea,naf->nef', qr, k) / jnp.sqrt(jnp.float32(a_sz))
    attn = jax.nn.softmax(qk, axis=1)
    v = values.reshape(n, e_num, a_sz)
    aw = jnp.einsum('nef,nfa->nea', attn, v)
    all_w = jnp.concatenate([aw, jnp.zeros_like(aw)], axis=-1)
    all_w = jax.nn.softmax(all_w, axis=-1)
    ar = all_w[:, :, :a_sz].sum(axis=-1)
    an = all_w[:, :, a_sz:].sum(axis=-1)
    mask = (ar - an) > 0

    scale_t = jnp.where(mask, ar, 0.0).T.reshape(e_num, n, 1)

    gwb = gate_w.astype(jnp.bfloat16)
    uwb = up_w.astype(jnp.bfloat16)
    dwb = down_w.astype(jnp.bfloat16)

    tb = min(512, n)
    nf = 2 if ff % 2 == 0 and ff >= 2048 else 1
    ftile = ff // nf
    out = pl.pallas_call(
        _moe_kernel,
        grid=(n // tb, e_num, nf),
        in_specs=[
            pl.BlockSpec((tb, d), lambda t, e, f: (t, 0)),
            pl.BlockSpec((1, d, ftile), lambda t, e, f: (e, 0, f)),
            pl.BlockSpec((1, d, ftile), lambda t, e, f: (e, 0, f)),
            pl.BlockSpec((1, ftile, d), lambda t, e, f: (e, f, 0)),
            pl.BlockSpec((1, tb, 1), lambda t, e, f: (e, t, 0)),
        ],
        out_specs=pl.BlockSpec((tb, d), lambda t, e, f: (t, 0)),
        out_shape=jax.ShapeDtypeStruct((n, d), jnp.float32),
        scratch_shapes=[pltpu.VMEM((tb, d), jnp.float32)],
        compiler_params=pltpu.CompilerParams(
            dimension_semantics=("arbitrary", "arbitrary", "arbitrary"),
        ),
    )(hs, gwb, uwb, dwb, scale_t)

    return out.reshape(b, s, d), ar, mask


# speedup vs baseline: 1.2785x; 1.0003x over previous
import jax
import jax.numpy as jnp
from jax.experimental import pallas as pl
from jax.experimental.pallas import tpu as pltpu


def _proj_kernel(x_ref, wk_ref, wv_ref, wsf_ref, wq_ref, k_ref, v_ref, q_ref):
    x = x_ref[...]
    k_ref[...] = jnp.dot(x, wk_ref[...], preferred_element_type=jnp.float32)
    v_ref[...] = jnp.dot(x, wv_ref[...], preferred_element_type=jnp.float32)
    sf = jnp.dot(x, wsf_ref[...], preferred_element_type=jnp.float32)
    q_ref[...] = jnp.dot(sf, wq_ref[...], preferred_element_type=jnp.float32)


def _moe_kernel(x_ref, gw_ref, uw_ref, dw_ref, scale_ref, out_ref, x_save_ref):
    e = pl.program_id(1)
    f = pl.program_id(2)

    @pl.when((e == 0) & (f == 0))
    def _():
        out_ref[...] = x_ref[...]

    @pl.when(f == 0)
    def _():
        x_save_ref[...] = out_ref[...]

    xb = x_save_ref[...].astype(jnp.bfloat16)
    g = jnp.dot(xb, gw_ref[0], preferred_element_type=jnp.float32)
    u = jnp.dot(xb, uw_ref[0], preferred_element_type=jnp.float32)
    h = (jax.nn.silu(g) * u).astype(jnp.bfloat16)
    o = jnp.dot(h, dw_ref[0], preferred_element_type=jnp.float32)
    out_ref[...] = out_ref[...] + o * scale_ref[0]


def kernel(hidden_states, Wk, Wv, Wq, Wsf, gate_w, up_w, down_w):
    b, s, d = hidden_states.shape
    n = b * s
    e_num, _, ff = gate_w.shape
    ea = Wk.shape[1]
    a_sz = ea // e_num
    hs = hidden_states.reshape(n, d)

    tp = min(512, n)
    keys, values, q = pl.pallas_call(
        _proj_kernel,
        grid=(n // tp,),
        in_specs=[
            pl.BlockSpec((tp, d), lambda t: (t, 0)),
            pl.BlockSpec((d, ea), lambda t: (0, 0)),
            pl.BlockSpec((d, ea), lambda t: (0, 0)),
            pl.BlockSpec((d, ea), lambda t: (0, 0)),
            pl.BlockSpec((ea, ea), lambda t: (0, 0)),
        ],
        out_specs=[pl.BlockSpec((tp, ea), lambda t: (t, 0))] * 3,
        out_shape=[jax.ShapeDtypeStruct((n, ea), jnp.float32)] * 3,
    )(hs, Wk, Wv, Wsf, Wq)

    qr = q.reshape(n, e_num, a_sz)
    k = keys.reshape(n, a_sz, e_num)
    qk = jnp.einsum('nea,naf->nef', qr, k) / jnp.sqrt(jnp.float32(a_sz))
    attn = jax.nn.softmax(qk, axis=1)
    v = values.reshape(n, e_num, a_sz)
    aw = jnp.einsum('nef,nfa->nea', attn, v)
    all_w = jnp.concatenate([aw, jnp.zeros_like(aw)], axis=-1)
    all_w = jax.nn.softmax(all_w, axis=-1)
    ar = all_w[:, :, :a_sz].sum(axis=-1)
    an = all_w[:, :, a_sz:].sum(axis=-1)
    mask = (ar - an) > 0

    scale_t = jnp.where(mask, ar, 0.0).T.reshape(e_num, n, 1)

    gwb = gate_w.astype(jnp.bfloat16)
    uwb = up_w.astype(jnp.bfloat16)
    dwb = down_w.astype(jnp.bfloat16)

    tb = min(512, n)
    nf = 2 if ff % 2 == 0 and ff >= 2048 else 1
    ftile = ff // nf
    out = pl.pallas_call(
        _moe_kernel,
        grid=(n // tb, e_num, nf),
        in_specs=[
            pl.BlockSpec((tb, d), lambda t, e, f: (t, 0)),
            pl.BlockSpec((1, d, ftile), lambda t, e, f: (e, 0, f)),
            pl.BlockSpec((1, d, ftile), lambda t, e, f: (e, 0, f)),
            pl.BlockSpec((1, ftile, d), lambda t, e, f: (e, f, 0)),
            pl.BlockSpec((1, tb, 1), lambda t, e, f: (e, t, 0)),
        ],
        out_specs=pl.BlockSpec((tb, d), lambda t, e, f: (t, 0)),
        out_shape=jax.ShapeDtypeStruct((n, d), jnp.float32),
        scratch_shapes=[pltpu.VMEM((tb, d), jnp.float32)],
        compiler_params=pltpu.CompilerParams(
            dimension_semantics=("parallel", "arbitrary", "arbitrary"),
        ),
    )(hs, gwb, uwb, dwb, scale_t)

    return out.reshape(b, s, d), ar, mask
